# baseline (speedup 1.0000x reference)
import functools

import jax
import jax.numpy as jnp
from jax import lax
from jax.experimental import pallas as pl
from jax.experimental.pallas import tpu as pltpu
from jax.experimental.pallas import tpu_sc as plsc

_NC = 2
_NS = 16
_CH = 128
_FRAC0 = 0.5


def _safe_inv(d):
    return jnp.where(d > 0, 1.0 / jnp.where(d > 0, d, 1.0), 0.0)


@functools.lru_cache(maxsize=None)
def _make_sc_phase(n_rows, n_acc, width, c_sc0, c_sc1):
    mesh = plsc.VectorSubcoreMesh(core_axis_name="c", subcore_axis_name="s")
    c_max = max(c_sc0, c_sc1)

    @functools.partial(
        pl.kernel,
        out_type=jax.ShapeDtypeStruct((_NC, n_rows, width), jnp.float32),
        mesh=mesh,
        scratch_types=[
            pltpu.VMEM((c_max, _CH), jnp.int32),
            pltpu.VMEM((c_max, _CH), jnp.int32),
            pltpu.VMEM((_CH, width), jnp.float32),
            pltpu.VMEM_SHARED((n_acc, width), jnp.float32),
            pltpu.SemaphoreType.DMA,
        ],
    )
    def phase(table, src_idx, dst_idx, zeros_hbm, out,
              idx_s, idx_d, rows, acc, gsem):
        cid = lax.axis_index("c")
        sid = lax.axis_index("s")
        zfull = -(-n_acc // (_NS * 8)) * 8
        zlast = n_acc - (_NS - 1) * zfull

        @pl.when(sid < _NS - 1)
        def _():
            pltpu.sync_copy(zeros_hbm.at[pl.ds(sid * zfull, zfull)],
                            acc.at[pl.ds(sid * zfull, zfull)])

        @pl.when(sid == _NS - 1)
        def _():
            pltpu.sync_copy(zeros_hbm.at[pl.ds((_NS - 1) * zfull, zlast)],
                            acc.at[pl.ds((_NS - 1) * zfull, zlast)])
        plsc.subcore_barrier()

        def run(start, count):
            pltpu.sync_copy(src_idx.at[pl.ds(start, count)],
                            idx_s.at[pl.ds(0, count)])
            pltpu.sync_copy(dst_idx.at[pl.ds(start, count)],
                            idx_d.at[pl.ds(0, count)])

            def body(j, carry):
                pltpu.async_copy(table.at[idx_s.at[j]], rows, gsem).wait()
                pltpu.sync_copy(rows, acc.at[idx_d.at[j]], add=True)
                return carry

            lax.fori_loop(0, count, body, 0)

        @pl.when(cid == 0)
        def _():
            run(sid * c_sc0, c_sc0)

        @pl.when(cid == 1)
        def _():
            run(_NS * c_sc0 + sid * c_sc1, c_sc1)

        plsc.subcore_barrier()
        full = -(-n_rows // (_NS * 8)) * 8
        last = n_rows - (_NS - 1) * full

        @pl.when(sid < _NS - 1)
        def _():
            pltpu.sync_copy(acc.at[pl.ds(sid * full, full)],
                            out.at[cid, pl.ds(sid * full, full)])

        @pl.when(sid == _NS - 1)
        def _():
            pltpu.sync_copy(acc.at[pl.ds((_NS - 1) * full, last)],
                            out.at[cid, pl.ds((_NS - 1) * full, last)])

    return phase


def _tc_matmul_ones(x, wp):
    m, k = x.shape
    wd = wp.shape[1]
    bm = 1000

    def body(x_ref, w_ref, o_ref):
        acc = jnp.dot(x_ref[...], w_ref[...], preferred_element_type=jnp.float32)
        col = lax.broadcasted_iota(jnp.int32, (1, wd), 1)
        o_ref[...] = acc + (col == 64).astype(jnp.float32)

    return pl.pallas_call(
        body,
        grid=(m // bm,),
        in_specs=[pl.BlockSpec((bm, k), lambda i: (i, 0)),
                  pl.BlockSpec((k, wd), lambda i: (0, 0))],
        out_specs=pl.BlockSpec((bm, wd), lambda i: (i, 0)),
        out_shape=jax.ShapeDtypeStruct((m, wd), jnp.float32),
    )(x, wp)


def _tc_scale(a0, a1, c0, c1):
    m, wd = a0.shape
    bm = 1000

    def body(a0r, a1r, c0r, c1r, o_ref):
        s = a0r[...] + a1r[...]
        inv = _safe_inv(c0r[:, 64:65] + c1r[:, 64:65])
        o_ref[...] = inv * s

    spec = pl.BlockSpec((bm, wd), lambda i: (i, 0))
    return pl.pallas_call(
        body,
        grid=(m // bm,),
        in_specs=[spec, spec, spec, spec],
        out_specs=spec,
        out_shape=jax.ShapeDtypeStruct((m, wd), jnp.float32),
    )(a0, a1, c0, c1)


def _tc_hidden(p0, p1, b1r, w2p):
    m, wd = p0.shape
    h = b1r.shape[1]
    bm = 1000

    def body(p0r, p1r, br, wr, o_ref):
        s = p0r[...] + p1r[...]
        inv = _safe_inv(s[:, 64:65])
        hid = jnp.maximum(inv * s[:, :h] + br[...], 0.0)
        o_ref[...] = jnp.dot(hid, wr[...], preferred_element_type=jnp.float32)

    return pl.pallas_call(
        body,
        grid=(m // bm,),
        in_specs=[pl.BlockSpec((bm, wd), lambda i: (i, 0)),
                  pl.BlockSpec((bm, wd), lambda i: (i, 0)),
                  pl.BlockSpec((1, h), lambda i: (0, 0)),
                  pl.BlockSpec((h, wd), lambda i: (0, 0))],
        out_specs=pl.BlockSpec((bm, wd), lambda i: (i, 0)),
        out_shape=jax.ShapeDtypeStruct((m, wd), jnp.float32),
    )(p0, p1, b1r, w2p)


def _tc_final(d0, d1, p0, p1, b2r):
    m, wd = p0.shape
    bm = 1000

    def body(d0r, d1r, p0r, p1r, br, o_ref):
        inv = _safe_inv(d0r[:, 64:65] + d1r[:, 64:65])
        o_ref[...] = inv * (p0r[...] + p1r[...]) + br[...]

    spec = pl.BlockSpec((bm, wd), lambda i: (i, 0))
    return pl.pallas_call(
        body,
        grid=(m // bm,),
        in_specs=[spec, spec, spec, spec,
                  pl.BlockSpec((1, wd), lambda i: (0, 0))],
        out_specs=spec,
        out_shape=jax.ShapeDtypeStruct((m, wd), jnp.float32),
    )(d0, d1, p0, p1, b2r)


def kernel(x, hyperedge_index, W1, b1, W2, b2):
    n, f = x.shape
    h = W1.shape[1]
    c = W2.shape[1]
    e = hyperedge_index.shape[1]
    wd = 128

    per_pair = -(-(-(-e // (_CH * _NS))) // 8) * 8
    c_sc0 = int(round(per_pair * _FRAC0 / 8)) * 8
    c_sc1 = per_pair - c_sc0
    n_chunks = per_pair * _NS
    ep = n_chunks * _CH
    n_acc = -(-(n + 1) // 8) * 8 + 248

    idx0 = hyperedge_index[0].astype(jnp.int32)
    idx1 = hyperedge_index[1].astype(jnp.int32)
    pad_s = jnp.zeros((ep - e,), jnp.int32)
    pad_d = n + jnp.arange(ep - e, dtype=jnp.int32) % (n_acc - n)
    src_a = jnp.concatenate([idx0, pad_s]).reshape(n_chunks, _CH)
    dst_a = jnp.concatenate([idx1, pad_d]).reshape(n_chunks, _CH)
    src_b = jnp.concatenate([idx1, pad_s]).reshape(n_chunks, _CH)
    dst_b = jnp.concatenate([idx0, pad_d]).reshape(n_chunks, _CH)

    w1p = jnp.pad(W1, ((0, 0), (0, wd - h)))
    w2p = jnp.pad(W2, ((0, 0), (0, wd - c)))
    b1r = b1.reshape(1, h)
    b2r = jnp.pad(b2, (0, wd - c)).reshape(1, wd)
    zeros = jnp.zeros((n_acc, wd), jnp.float32)

    phase = _make_sc_phase(n, n_acc, wd, c_sc0, c_sc1)

    xw1 = _tc_matmul_ones(x, w1p)
    s1 = phase(xw1, src_a, dst_a, zeros)
    e1 = _tc_scale(s1[0], s1[1], s1[0], s1[1])
    s2 = phase(e1, src_b, dst_b, zeros)
    xw2 = _tc_hidden(s2[0], s2[1], b1r, w2p)
    s3 = phase(xw2, src_a, dst_a, zeros)
    e2 = _tc_scale(s3[0], s3[1], s1[0], s1[1])
    s4 = phase(e2, src_b, dst_b, zeros)
    out = _tc_final(s2[0], s2[1], s4[0], s4[1], b2r)
    return out[:, :c]


# speedup vs baseline: 1.2667x; 1.2667x over previous
import functools

import jax
import jax.numpy as jnp
from jax import lax
from jax.experimental import pallas as pl
from jax.experimental.pallas import tpu as pltpu
from jax.experimental.pallas import tpu_sc as plsc

_NC = 2
_NS = 16
_CH = 128
_FRAC0 = 0.5


def _safe_inv(d):
    return jnp.where(d > 0, 1.0 / jnp.where(d > 0, d, 1.0), 0.0)


@functools.lru_cache(maxsize=None)
def _make_sc_phase(n_rows, n_acc, width, c_sc0, c_sc1):
    mesh = plsc.VectorSubcoreMesh(core_axis_name="c", subcore_axis_name="s")
    c_max = max(c_sc0, c_sc1)

    @functools.partial(
        pl.kernel,
        out_type=jax.ShapeDtypeStruct((_NC, n_rows, width), jnp.float32),
        mesh=mesh,
        scratch_types=[
            pltpu.VMEM((c_max, _CH), jnp.int32),
            pltpu.VMEM((c_max, _CH), jnp.int32),
            pltpu.VMEM((_CH, width), jnp.float32),
            pltpu.VMEM_SHARED((n_acc, width), jnp.float32),
            pltpu.SemaphoreType.DMA,
        ],
        compiler_params=pltpu.CompilerParams(use_tc_tiling_on_sc=False),
    )
    def phase(table, src_idx, dst_idx, zeros_hbm, out,
              idx_s, idx_d, rows, acc, gsem):
        cid = lax.axis_index("c")
        sid = lax.axis_index("s")
        zfull = -(-n_acc // (_NS * 8)) * 8
        zlast = n_acc - (_NS - 1) * zfull

        @pl.when(sid < _NS - 1)
        def _():
            pltpu.sync_copy(zeros_hbm.at[pl.ds(sid * zfull, zfull)],
                            acc.at[pl.ds(sid * zfull, zfull)])

        @pl.when(sid == _NS - 1)
        def _():
            pltpu.sync_copy(zeros_hbm.at[pl.ds((_NS - 1) * zfull, zlast)],
                            acc.at[pl.ds((_NS - 1) * zfull, zlast)])
        plsc.subcore_barrier()

        def run(start, count):
            pltpu.sync_copy(src_idx.at[pl.ds(start, count)],
                            idx_s.at[pl.ds(0, count)])
            pltpu.sync_copy(dst_idx.at[pl.ds(start, count)],
                            idx_d.at[pl.ds(0, count)])

            def body(j, carry):
                pltpu.async_copy(table.at[idx_s.at[j]], rows, gsem).wait()
                pltpu.sync_copy(rows, acc.at[idx_d.at[j]], add=True)
                return carry

            lax.fori_loop(0, count, body, 0)

        @pl.when(cid == 0)
        def _():
            run(sid * c_sc0, c_sc0)

        @pl.when(cid == 1)
        def _():
            run(_NS * c_sc0 + sid * c_sc1, c_sc1)

        plsc.subcore_barrier()
        full = -(-n_rows // (_NS * 8)) * 8
        last = n_rows - (_NS - 1) * full

        @pl.when(sid < _NS - 1)
        def _():
            pltpu.sync_copy(acc.at[pl.ds(sid * full, full)],
                            out.at[cid, pl.ds(sid * full, full)])

        @pl.when(sid == _NS - 1)
        def _():
            pltpu.sync_copy(acc.at[pl.ds((_NS - 1) * full, last)],
                            out.at[cid, pl.ds((_NS - 1) * full, last)])

    return phase


def _tc_matmul_ones(x, wp):
    m, k = x.shape
    wd = wp.shape[1]
    bm = 1000

    def body(x_ref, w_ref, o_ref):
        acc = jnp.dot(x_ref[...], w_ref[...], preferred_element_type=jnp.float32)
        col = lax.broadcasted_iota(jnp.int32, (1, wd), 1)
        o_ref[...] = acc + (col == 64).astype(jnp.float32)

    return pl.pallas_call(
        body,
        grid=(m // bm,),
        in_specs=[pl.BlockSpec((bm, k), lambda i: (i, 0)),
                  pl.BlockSpec((k, wd), lambda i: (0, 0))],
        out_specs=pl.BlockSpec((bm, wd), lambda i: (i, 0)),
        out_shape=jax.ShapeDtypeStruct((m, wd), jnp.float32),
    )(x, wp)


def _tc_scale(a0, a1, c0, c1):
    m, wd = a0.shape
    bm = 1000

    def body(a0r, a1r, c0r, c1r, o_ref):
        s = a0r[...] + a1r[...]
        inv = _safe_inv(c0r[:, 64:65] + c1r[:, 64:65])
        o_ref[...] = inv * s

    spec = pl.BlockSpec((bm, wd), lambda i: (i, 0))
    return pl.pallas_call(
        body,
        grid=(m // bm,),
        in_specs=[spec, spec, spec, spec],
        out_specs=spec,
        out_shape=jax.ShapeDtypeStruct((m, wd), jnp.float32),
    )(a0, a1, c0, c1)


def _tc_hidden(p0, p1, b1r, w2p):
    m, wd = p0.shape
    h = b1r.shape[1]
    bm = 1000

    def body(p0r, p1r, br, wr, o_ref):
        s = p0r[...] + p1r[...]
        inv = _safe_inv(s[:, 64:65])
        hid = jnp.maximum(inv * s[:, :h] + br[...], 0.0)
        o_ref[...] = jnp.dot(hid, wr[...], preferred_element_type=jnp.float32)

    return pl.pallas_call(
        body,
        grid=(m // bm,),
        in_specs=[pl.BlockSpec((bm, wd), lambda i: (i, 0)),
                  pl.BlockSpec((bm, wd), lambda i: (i, 0)),
                  pl.BlockSpec((1, h), lambda i: (0, 0)),
                  pl.BlockSpec((h, wd), lambda i: (0, 0))],
        out_specs=pl.BlockSpec((bm, wd), lambda i: (i, 0)),
        out_shape=jax.ShapeDtypeStruct((m, wd), jnp.float32),
    )(p0, p1, b1r, w2p)


def _tc_final(d0, d1, p0, p1, b2r):
    m, wd = p0.shape
    bm = 1000

    def body(d0r, d1r, p0r, p1r, br, o_ref):
        inv = _safe_inv(d0r[:, 64:65] + d1r[:, 64:65])
        o_ref[...] = inv * (p0r[...] + p1r[...]) + br[...]

    spec = pl.BlockSpec((bm, wd), lambda i: (i, 0))
    return pl.pallas_call(
        body,
        grid=(m // bm,),
        in_specs=[spec, spec, spec, spec,
                  pl.BlockSpec((1, wd), lambda i: (0, 0))],
        out_specs=spec,
        out_shape=jax.ShapeDtypeStruct((m, wd), jnp.float32),
    )(d0, d1, p0, p1, b2r)


def kernel(x, hyperedge_index, W1, b1, W2, b2):
    n, f = x.shape
    h = W1.shape[1]
    c = W2.shape[1]
    e = hyperedge_index.shape[1]
    wd = 80

    per_pair = -(-(-(-e // (_CH * _NS))) // 8) * 8
    c_sc0 = int(round(per_pair * _FRAC0 / 8)) * 8
    c_sc1 = per_pair - c_sc0
    n_chunks = per_pair * _NS
    ep = n_chunks * _CH
    n_acc = -(-(n + 1) // 8) * 8 + 248

    idx0 = hyperedge_index[0].astype(jnp.int32)
    idx1 = hyperedge_index[1].astype(jnp.int32)
    pad_s = jnp.zeros((ep - e,), jnp.int32)
    pad_d = n + jnp.arange(ep - e, dtype=jnp.int32) % (n_acc - n)
    src_a = jnp.concatenate([idx0, pad_s]).reshape(n_chunks, _CH)
    dst_a = jnp.concatenate([idx1, pad_d]).reshape(n_chunks, _CH)
    src_b = jnp.concatenate([idx1, pad_s]).reshape(n_chunks, _CH)
    dst_b = jnp.concatenate([idx0, pad_d]).reshape(n_chunks, _CH)

    w1p = jnp.pad(W1, ((0, 0), (0, wd - h)))
    w2p = jnp.pad(W2, ((0, 0), (0, wd - c)))
    b1r = b1.reshape(1, h)
    b2r = jnp.pad(b2, (0, wd - c)).reshape(1, wd)
    zeros = jnp.zeros((n_acc, wd), jnp.float32)

    phase = _make_sc_phase(n, n_acc, wd, c_sc0, c_sc1)

    xw1 = _tc_matmul_ones(x, w1p)
    s1 = phase(xw1, src_a, dst_a, zeros)
    e1 = _tc_scale(s1[0], s1[1], s1[0], s1[1])
    s2 = phase(e1, src_b, dst_b, zeros)
    xw2 = _tc_hidden(s2[0], s2[1], b1r, w2p)
    s3 = phase(xw2, src_a, dst_a, zeros)
    e2 = _tc_scale(s3[0], s3[1], s1[0], s1[1])
    s4 = phase(e2, src_b, dst_b, zeros)
    out = _tc_final(s2[0], s2[1], s4[0], s4[1], b2r)
    return out[:, :c]


# speedup vs baseline: 1.5717x; 1.2408x over previous
import functools

import jax
import jax.numpy as jnp
from jax import lax
from jax.experimental import pallas as pl
from jax.experimental.pallas import tpu as pltpu
from jax.experimental.pallas import tpu_sc as plsc

_NC = 2
_NS = 16
_CH = 128


def _safe_inv(d):
    return jnp.where(d > 0, 1.0 / jnp.where(d > 0, d, 1.0), 0.0)


@functools.lru_cache(maxsize=None)
def _make_sc_phase(n_rows, n_acc, width, n_per_tile):
    mesh = plsc.VectorSubcoreMesh(core_axis_name="c", subcore_axis_name="s")

    @functools.partial(
        pl.kernel,
        out_type=jax.ShapeDtypeStruct((_NC, n_rows, width), jnp.float32),
        mesh=mesh,
        scratch_types=[
            pltpu.VMEM((n_per_tile, _CH), jnp.int32),
            pltpu.VMEM((n_per_tile, _CH), jnp.int32),
            pltpu.VMEM((_CH, width), jnp.float32),
            pltpu.VMEM_SHARED((n_acc, width), jnp.float32),
            pltpu.SemaphoreType.DMA,
        ],
        compiler_params=pltpu.CompilerParams(use_tc_tiling_on_sc=False),
    )
    def phase(table, src_idx, dst_idx, zeros_hbm, out,
              idx_s, idx_d, rows, acc, gsem):
        cid = lax.axis_index("c")
        sid = lax.axis_index("s")
        zfull = -(-n_acc // (_NS * 8)) * 8
        zlast = n_acc - (_NS - 1) * zfull

        @pl.when(sid < _NS - 1)
        def _():
            pltpu.sync_copy(zeros_hbm.at[pl.ds(sid * zfull, zfull)],
                            acc.at[pl.ds(sid * zfull, zfull)])

        @pl.when(sid == _NS - 1)
        def _():
            pltpu.sync_copy(zeros_hbm.at[pl.ds((_NS - 1) * zfull, zlast)],
                            acc.at[pl.ds((_NS - 1) * zfull, zlast)])
        wid = cid * _NS + sid
        pltpu.sync_copy(src_idx.at[wid], idx_s)
        pltpu.sync_copy(dst_idx.at[wid], idx_d)
        plsc.subcore_barrier()

        def body(j, carry):
            pltpu.async_copy(table.at[idx_s.at[j]], rows, gsem).wait()
            pltpu.sync_copy(rows, acc.at[idx_d.at[j]], add=True)
            return carry

        lax.fori_loop(0, n_per_tile, body, 0)

        plsc.subcore_barrier()
        full = -(-n_rows // (_NS * 8)) * 8
        last = n_rows - (_NS - 1) * full

        @pl.when(sid < _NS - 1)
        def _():
            pltpu.sync_copy(acc.at[pl.ds(sid * full, full)],
                            out.at[cid, pl.ds(sid * full, full)])

        @pl.when(sid == _NS - 1)
        def _():
            pltpu.sync_copy(acc.at[pl.ds((_NS - 1) * full, last)],
                            out.at[cid, pl.ds((_NS - 1) * full, last)])

    return phase


def _tc_matmul_ones(x, wp):
    m, k = x.shape
    wd = wp.shape[1]
    bm = 1000

    def body(x_ref, w_ref, o_ref):
        acc = jnp.dot(x_ref[...], w_ref[...], preferred_element_type=jnp.float32)
        col = lax.broadcasted_iota(jnp.int32, (1, wd), 1)
        o_ref[...] = acc + (col == 64).astype(jnp.float32)

    return pl.pallas_call(
        body,
        grid=(m // bm,),
        in_specs=[pl.BlockSpec((bm, k), lambda i: (i, 0)),
                  pl.BlockSpec((k, wd), lambda i: (0, 0))],
        out_specs=pl.BlockSpec((bm, wd), lambda i: (i, 0)),
        out_shape=jax.ShapeDtypeStruct((m, wd), jnp.float32),
    )(x, wp)


def _tc_scale(a0, a1, c0, c1):
    m, wd = a0.shape
    bm = 1000

    def body(a0r, a1r, c0r, c1r, o_ref):
        s = a0r[...] + a1r[...]
        inv = _safe_inv(c0r[:, 64:65] + c1r[:, 64:65])
        o_ref[...] = inv * s

    spec = pl.BlockSpec((bm, wd), lambda i: (i, 0))
    return pl.pallas_call(
        body,
        grid=(m // bm,),
        in_specs=[spec, spec, spec, spec],
        out_specs=spec,
        out_shape=jax.ShapeDtypeStruct((m, wd), jnp.float32),
    )(a0, a1, c0, c1)


def _tc_hidden(p0, p1, b1r, w2p):
    m, wd = p0.shape
    h = b1r.shape[1]
    bm = 1000

    def body(p0r, p1r, br, wr, o_ref):
        s = p0r[...] + p1r[...]
        inv = _safe_inv(s[:, 64:65])
        hid = jnp.maximum(inv * s[:, :h] + br[...], 0.0)
        o_ref[...] = jnp.dot(hid, wr[...], preferred_element_type=jnp.float32)

    return pl.pallas_call(
        body,
        grid=(m // bm,),
        in_specs=[pl.BlockSpec((bm, wd), lambda i: (i, 0)),
                  pl.BlockSpec((bm, wd), lambda i: (i, 0)),
                  pl.BlockSpec((1, h), lambda i: (0, 0)),
                  pl.BlockSpec((h, wd), lambda i: (0, 0))],
        out_specs=pl.BlockSpec((bm, wd), lambda i: (i, 0)),
        out_shape=jax.ShapeDtypeStruct((m, wd), jnp.float32),
    )(p0, p1, b1r, w2p)


def _tc_final(d0, d1, p0, p1, b2r):
    m, wd = p0.shape
    bm = 1000

    def body(d0r, d1r, p0r, p1r, br, o_ref):
        inv = _safe_inv(d0r[:, 64:65] + d1r[:, 64:65])
        o_ref[...] = inv * (p0r[...] + p1r[...]) + br[...]

    spec = pl.BlockSpec((bm, wd), lambda i: (i, 0))
    return pl.pallas_call(
        body,
        grid=(m // bm,),
        in_specs=[spec, spec, spec, spec,
                  pl.BlockSpec((1, wd), lambda i: (0, 0))],
        out_specs=spec,
        out_shape=jax.ShapeDtypeStruct((m, wd), jnp.float32),
    )(d0, d1, p0, p1, b2r)


def kernel(x, hyperedge_index, W1, b1, W2, b2):
    n, f = x.shape
    h = W1.shape[1]
    c = W2.shape[1]
    e = hyperedge_index.shape[1]
    wd = 80

    nwk = _NC * _NS
    n_per_tile = -(-e // (nwk * _CH))
    ep = nwk * n_per_tile * _CH
    n_acc = -(-(n + 1) // 8) * 8 + 504

    idx0 = hyperedge_index[0].astype(jnp.int32)
    idx1 = hyperedge_index[1].astype(jnp.int32)
    pad_s = jnp.zeros((ep - e,), jnp.int32)
    pad_d = n + jnp.arange(ep - e, dtype=jnp.int32) % (n_acc - n)
    src_a = jnp.concatenate([idx0, pad_s]).reshape(nwk, n_per_tile, _CH)
    dst_a = jnp.concatenate([idx1, pad_d]).reshape(nwk, n_per_tile, _CH)
    src_b = jnp.concatenate([idx1, pad_s]).reshape(nwk, n_per_tile, _CH)
    dst_b = jnp.concatenate([idx0, pad_d]).reshape(nwk, n_per_tile, _CH)

    w1p = jnp.pad(W1, ((0, 0), (0, wd - h)))
    w2p = jnp.pad(W2, ((0, 0), (0, wd - c)))
    b1r = b1.reshape(1, h)
    b2r = jnp.pad(b2, (0, wd - c)).reshape(1, wd)
    zeros = jnp.zeros((n_acc, wd), jnp.float32)

    phase = _make_sc_phase(n, n_acc, wd, n_per_tile)

    xw1 = _tc_matmul_ones(x, w1p)
    s1 = phase(xw1, src_a, dst_a, zeros)
    e1 = _tc_scale(s1[0], s1[1], s1[0], s1[1])
    s2 = phase(e1, src_b, dst_b, zeros)
    xw2 = _tc_hidden(s2[0], s2[1], b1r, w2p)
    s3 = phase(xw2, src_a, dst_a, zeros)
    e2 = _tc_scale(s3[0], s3[1], s1[0], s1[1])
    s4 = phase(e2, src_b, dst_b, zeros)
    out = _tc_final(s2[0], s2[1], s4[0], s4[1], b2r)
    return out[:, :c]


# speedup vs baseline: 2.0016x; 1.2735x over previous
import functools

import jax
import jax.numpy as jnp
from jax import lax
from jax.experimental import pallas as pl
from jax.experimental.pallas import tpu as pltpu
from jax.experimental.pallas import tpu_sc as plsc

_NC = 2
_NS = 16
_CH = 128


def _safe_inv(d):
    return jnp.where(d > 0, 1.0 / jnp.where(d > 0, d, 1.0), 0.0)


@functools.lru_cache(maxsize=None)
def _make_sc_phase(n_rows, n_acc, width, c_sc0, c_sc1):
    mesh = plsc.VectorSubcoreMesh(core_axis_name="c", subcore_axis_name="s")
    n_per_tile = max(c_sc0, c_sc1)

    @functools.partial(
        pl.kernel,
        out_type=jax.ShapeDtypeStruct((_NC, n_rows, width), jnp.float32),
        mesh=mesh,
        scratch_types=[
            pltpu.VMEM((n_per_tile, _CH), jnp.int32),
            pltpu.VMEM((n_per_tile, _CH), jnp.int32),
            pltpu.VMEM((_CH, width), jnp.float32),
            pltpu.VMEM_SHARED((n_acc, width), jnp.float32),
            pltpu.SemaphoreType.DMA,
        ],
        compiler_params=pltpu.CompilerParams(use_tc_tiling_on_sc=False),
    )
    def phase(table, src_idx, dst_idx, zeros_hbm, out,
              idx_s, idx_d, rows, acc, gsem):
        cid = lax.axis_index("c")
        sid = lax.axis_index("s")
        zfull = -(-n_acc // (_NS * 8)) * 8
        zlast = n_acc - (_NS - 1) * zfull

        @pl.when(sid < _NS - 1)
        def _():
            pltpu.sync_copy(zeros_hbm.at[pl.ds(sid * zfull, zfull)],
                            acc.at[pl.ds(sid * zfull, zfull)])

        @pl.when(sid == _NS - 1)
        def _():
            pltpu.sync_copy(zeros_hbm.at[pl.ds((_NS - 1) * zfull, zlast)],
                            acc.at[pl.ds((_NS - 1) * zfull, zlast)])
        wid = cid * _NS + sid
        pltpu.sync_copy(src_idx.at[wid], idx_s)
        pltpu.sync_copy(dst_idx.at[wid], idx_d)
        plsc.subcore_barrier()

        def body(j, carry):
            pltpu.async_copy(table.at[idx_s.at[j]], rows, gsem).wait()
            pltpu.sync_copy(rows, acc.at[idx_d.at[j]], add=True)
            return carry

        count = jnp.where(cid == 0, c_sc0, c_sc1)
        lax.fori_loop(0, count, body, 0)

        plsc.subcore_barrier()
        full = -(-n_rows // (_NS * 8)) * 8
        last = n_rows - (_NS - 1) * full

        @pl.when(sid < _NS - 1)
        def _():
            pltpu.sync_copy(acc.at[pl.ds(sid * full, full)],
                            out.at[cid, pl.ds(sid * full, full)])

        @pl.when(sid == _NS - 1)
        def _():
            pltpu.sync_copy(acc.at[pl.ds((_NS - 1) * full, last)],
                            out.at[cid, pl.ds((_NS - 1) * full, last)])

    return phase


def _tc_matmul_ones(x, wp):
    m, k = x.shape
    wd = wp.shape[1]
    bm = 1000

    def body(x_ref, w_ref, o_ref):
        acc = jnp.dot(x_ref[...], w_ref[...], preferred_element_type=jnp.float32)
        col = lax.broadcasted_iota(jnp.int32, (1, wd), 1)
        o_ref[...] = acc + (col == 64).astype(jnp.float32)

    return pl.pallas_call(
        body,
        grid=(m // bm,),
        in_specs=[pl.BlockSpec((bm, k), lambda i: (i, 0)),
                  pl.BlockSpec((k, wd), lambda i: (0, 0))],
        out_specs=pl.BlockSpec((bm, wd), lambda i: (i, 0)),
        out_shape=jax.ShapeDtypeStruct((m, wd), jnp.float32),
    )(x, wp)


def _tc_scale(a0, a1, c0, c1):
    m, wd = a0.shape
    bm = 1000

    def body(a0r, a1r, c0r, c1r, o_ref):
        s = a0r[...] + a1r[...]
        inv = _safe_inv(c0r[:, 64:65] + c1r[:, 64:65])
        o_ref[...] = inv * s

    spec = pl.BlockSpec((bm, wd), lambda i: (i, 0))
    return pl.pallas_call(
        body,
        grid=(m // bm,),
        in_specs=[spec, spec, spec, spec],
        out_specs=spec,
        out_shape=jax.ShapeDtypeStruct((m, wd), jnp.float32),
    )(a0, a1, c0, c1)


def _tc_hidden(p0, p1, b1r, w2p):
    m, wd = p0.shape
    h = b1r.shape[1]
    bm = 1000

    def body(p0r, p1r, br, wr, o_ref):
        s = p0r[...] + p1r[...]
        inv = _safe_inv(s[:, 64:65])
        hid = jnp.maximum(inv * s[:, :h] + br[...], 0.0)
        o_ref[...] = jnp.dot(hid, wr[...], preferred_element_type=jnp.float32)

    return pl.pallas_call(
        body,
        grid=(m // bm,),
        in_specs=[pl.BlockSpec((bm, wd), lambda i: (i, 0)),
                  pl.BlockSpec((bm, wd), lambda i: (i, 0)),
                  pl.BlockSpec((1, h), lambda i: (0, 0)),
                  pl.BlockSpec((h, wd), lambda i: (0, 0))],
        out_specs=pl.BlockSpec((bm, wd), lambda i: (i, 0)),
        out_shape=jax.ShapeDtypeStruct((m, wd), jnp.float32),
    )(p0, p1, b1r, w2p)


def _tc_final(d0, d1, p0, p1, b2r):
    m, wd = p0.shape
    bm = 1000

    def body(d0r, d1r, p0r, p1r, br, o_ref):
        inv = _safe_inv(d0r[:, 64:65] + d1r[:, 64:65])
        o_ref[...] = inv * (p0r[...] + p1r[...]) + br[...]

    spec = pl.BlockSpec((bm, wd), lambda i: (i, 0))
    return pl.pallas_call(
        body,
        grid=(m // bm,),
        in_specs=[spec, spec, spec, spec,
                  pl.BlockSpec((1, wd), lambda i: (0, 0))],
        out_specs=spec,
        out_shape=jax.ShapeDtypeStruct((m, wd), jnp.float32),
    )(d0, d1, p0, p1, b2r)


def kernel(x, hyperedge_index, W1, b1, W2, b2):
    n, f = x.shape
    h = W1.shape[1]
    c = W2.shape[1]
    e = hyperedge_index.shape[1]
    wd = 80

    nwk = _NC * _NS
    c_pair = 2 * (-(-e // (nwk * _CH)))
    c_sc0 = int(round(c_pair * 2.0 / 3.0))
    c_sc1 = c_pair - c_sc0
    ep = _NS * c_pair * _CH
    n_acc = -(-(n + 1) // 8) * 8 + 504

    idx0 = hyperedge_index[0].astype(jnp.int32)
    idx1 = hyperedge_index[1].astype(jnp.int32)
    pad_s = jnp.zeros((ep - e,), jnp.int32)
    pad_d = n + jnp.arange(ep - e, dtype=jnp.int32) % (n_acc - n)
    def _slab(a, pad):
        flat = jnp.concatenate([a, pad]).reshape(-1, _CH)
        sc0 = flat[:_NS * c_sc0].reshape(_NS, c_sc0, _CH)
        sc1 = flat[_NS * c_sc0:].reshape(_NS, c_sc1, _CH)
        slab = jnp.zeros((nwk, c_sc0, _CH), jnp.int32)
        return slab.at[:_NS].set(sc0).at[_NS:, :c_sc1].set(sc1)

    src_a = _slab(idx0, pad_s)
    dst_a = _slab(idx1, pad_d)
    src_b = _slab(idx1, pad_s)
    dst_b = _slab(idx0, pad_d)

    w1p = jnp.pad(W1, ((0, 0), (0, wd - h)))
    w2p = jnp.pad(W2, ((0, 0), (0, wd - c)))
    b1r = b1.reshape(1, h)
    b2r = jnp.pad(b2, (0, wd - c)).reshape(1, wd)
    zeros = jnp.zeros((n_acc, wd), jnp.float32)

    phase = _make_sc_phase(n, n_acc, wd, c_sc0, c_sc1)

    xw1 = _tc_matmul_ones(x, w1p)
    s1 = phase(xw1, src_a, dst_a, zeros)
    e1 = _tc_scale(s1[0], s1[1], s1[0], s1[1])
    s2 = phase(e1, src_b, dst_b, zeros)
    xw2 = _tc_hidden(s2[0], s2[1], b1r, w2p)
    s3 = phase(xw2, src_a, dst_a, zeros)
    e2 = _tc_scale(s3[0], s3[1], s1[0], s1[1])
    s4 = phase(e2, src_b, dst_b, zeros)
    out = _tc_final(s2[0], s2[1], s4[0], s4[1], b2r)
    return out[:, :c]
